# baseline (speedup 1.0000x reference)
import functools

import jax
import jax.numpy as jnp
from jax import lax
from jax.experimental import pallas as pl
from jax.experimental.pallas import tpu as pltpu
from jax.experimental.pallas import tpu_sc as plsc

_LANES = 128
_QROWS = 16
_TILES = 16
_CORES = 2


def _make_sc_agg(n, d, np_, ep, with_cnt):
    rows_per_tile = ep // (_CORES * _TILES) // _LANES
    slc = np_ // _TILES
    nblk = d // 32

    mesh = plsc.VectorSubcoreMesh(core_axis_name="c", subcore_axis_name="s")

    out_types = [jax.ShapeDtypeStruct((_CORES * np_, d), jnp.float32)]
    half = rows_per_tile // 2
    scratch = [
        pltpu.VMEM((half, _LANES), jnp.int32),
        pltpu.VMEM((2 * half, _LANES // 2), jnp.int32),
        pltpu.VMEM((2, _LANES, d // 2), jnp.int32),
        pltpu.VMEM((2, _LANES // 2, d), jnp.float32),
        pltpu.VMEM_SHARED((np_, d), jnp.float32),
        pltpu.SemaphoreType.DMA,
        pltpu.SemaphoreType.DMA,
    ]
    if with_cnt:
        out_types.append(jax.ShapeDtypeStruct((_CORES * np_,), jnp.float32))
        scratch += [
            pltpu.VMEM((_LANES // 2,), jnp.float32),
            pltpu.VMEM_SHARED((np_,), jnp.float32),
            pltpu.VMEM((slc,), jnp.float32),
        ]

    def body(h, src2, dst2, zer2, zer1, ones1, p_out, cnt_out, sidx, didx,
             ring, fbuf, acc, sem, sem_s, ones_v, cntacc, cz):
        c = lax.axis_index("c")
        s = lax.axis_index("s")
        base = s * slc
        pltpu.sync_copy(zer2.at[pl.ds(base, slc)], acc.at[pl.ds(base, slc)])
        if with_cnt:
            pltpu.sync_copy(zer1.at[pl.ds(base, slc)], cz)
            pltpu.sync_copy(cz, cntacc.at[pl.ds(base, slc)])
            pltpu.sync_copy(ones1, ones_v)
        wrow = (c * _TILES + s) * rows_per_tile
        plsc.subcore_barrier()

        def convert(par, bufi, lo_row):
            def cb_loop(cb, carry):
                ci = pl.multiple_of(cb * 16, 16)
                co = pl.multiple_of(cb * 32, 32)
                ch = pl.multiple_of(cb * 32 + 16, 16)
                for r in range(_LANES // 2):
                    w = ring[par, lo_row + r, pl.ds(ci, 16)]
                    fbuf[bufi, r, pl.ds(co, 16)] = plsc.bitcast(
                        lax.shift_left(w, 16), jnp.float32)
                    fbuf[bufi, r, pl.ds(ch, 16)] = plsc.bitcast(
                        lax.shift_left(
                            lax.shift_right_arithmetic(w, 16), 16),
                        jnp.float32)
                return carry
            lax.fori_loop(0, nblk, cb_loop, 0)

        def group(j, par):
            pltpu.make_async_copy(h.at[sidx.at[j]], ring.at[par], sem).wait()

            @pl.when(j >= 1)
            def _():
                jm = lax.max(j - 1, 0)
                pltpu.make_async_copy(fbuf.at[0], acc.at[didx.at[2 * jm]],
                                      sem_s).wait()
                pltpu.make_async_copy(fbuf.at[1], acc.at[didx.at[2 * jm + 1]],
                                      sem_s).wait()

            convert(par, 0, 0)
            pltpu.async_copy(fbuf.at[0], acc.at[didx.at[2 * j]], sem_s,
                             add=True)
            convert(par, 1, _LANES // 2)
            pltpu.async_copy(fbuf.at[1], acc.at[didx.at[2 * j + 1]], sem_s,
                             add=True)
            if with_cnt:
                pltpu.sync_copy(ones_v, cntacc.at[didx.at[2 * j]], add=True)
                pltpu.sync_copy(ones_v, cntacc.at[didx.at[2 * j + 1]],
                                add=True)

            @pl.when(j + 2 < half)
            def _():
                pltpu.async_copy(h.at[sidx.at[j + 2]], ring.at[par], sem)

        for hh in range(2):
            hrow = wrow + hh * half
            pltpu.sync_copy(src2.at[pl.ds(hrow, half)], sidx)
            pltpu.sync_copy(dst2.at[pl.ds(2 * hrow, 2 * half)], didx)
            pltpu.async_copy(h.at[sidx.at[0]], ring.at[0], sem)
            pltpu.async_copy(h.at[sidx.at[1]], ring.at[1], sem)

            def jb2(jj, carry2):
                group(2 * jj, 0)
                group(2 * jj + 1, 1)
                return carry2

            lax.fori_loop(0, half // 2, jb2, 0)
            pltpu.make_async_copy(fbuf.at[0], acc.at[didx.at[2 * half - 2]],
                                  sem_s).wait()
            pltpu.make_async_copy(fbuf.at[1], acc.at[didx.at[2 * half - 1]],
                                  sem_s).wait()
        plsc.subcore_barrier()

        pltpu.sync_copy(acc.at[pl.ds(base, slc)],
                        p_out.at[pl.ds(c * np_ + base, slc)])
        if with_cnt:
            pltpu.sync_copy(cntacc.at[pl.ds(base, slc)], cz)
            pltpu.sync_copy(cz, cnt_out.at[pl.ds(c * np_ + base, slc)])

    if with_cnt:
        fn = body
    else:
        def fn(h, src2, dst2, zer2, p_out, sidx, didx, ring, fbuf, acc, sem,
               sem_s):
            return body(h, src2, dst2, zer2, None, None, p_out, None, sidx,
                        didx, ring, fbuf, acc, sem, sem_s, None, None, None)

    return pl.kernel(
        fn, mesh=mesh, out_type=out_types, scratch_types=scratch,
        compiler_params=pltpu.CompilerParams(use_tc_tiling_on_sc=False, needs_layout_passes=False))


def _r_call(h, wr):
    n, d = h.shape

    def body(h_ref, wr_ref, out_ref):
        out_ref[...] = jnp.dot(h_ref[...], wr_ref[...],
                               preferred_element_type=jnp.float32)

    return pl.pallas_call(
        body, out_shape=jax.ShapeDtypeStruct((n, d), jnp.float32))(h, wr)


def _combine_body(r_ref, p_ref, inv_ref, wl_ref, bl_ref, g_ref, b_ref,
                  out_ref, *, n, np_, final, wo_ref=None, bo_ref=None):
    agg = p_ref[0:n] + p_ref[np_:np_ + n]
    mean = agg * inv_ref[...]
    y = jnp.dot(mean, wl_ref[...], preferred_element_type=jnp.float32)
    y = y + bl_ref[...] + r_ref[...]
    mu = jnp.mean(y, axis=0, keepdims=True)
    var = jnp.mean((y - mu) * (y - mu), axis=0, keepdims=True)
    y = (y - mu) * lax.rsqrt(var + 1e-5) * g_ref[...] + b_ref[...]
    h = jnp.maximum(y, 0.0)
    if final:
        out_ref[...] = jnp.dot(h, wo_ref[...],
                               preferred_element_type=jnp.float32) + bo_ref[...]
    else:
        out_ref[...] = h


def _mid_layer_call(r, p, inv, wl, bl, g, b, n, d, np_):
    def body(r_ref, p_ref, inv_ref, wl_ref, bl_ref, g_ref, b_ref, out_ref,
             hb_ref):
        _combine_body(r_ref, p_ref, inv_ref, wl_ref, bl_ref, g_ref, b_ref,
                      out_ref, n=n, np_=np_, final=False)
        hb_ref[...] = out_ref[...].astype(jnp.bfloat16)
    return pl.pallas_call(
        body,
        out_shape=[jax.ShapeDtypeStruct((n, d), jnp.float32),
                   jax.ShapeDtypeStruct((n, d), jnp.bfloat16)],
    )(r, p, inv, wl, bl.reshape(1, d), g.reshape(1, d), b.reshape(1, d))


def _final_layer_call(r, p, inv, wl, bl, g, b, wo, bo, n, d, np_):
    def body(r_ref, p_ref, inv_ref, wl_ref, bl_ref, g_ref, b_ref, wo_ref,
             bo_ref, out_ref):
        _combine_body(r_ref, p_ref, inv_ref, wl_ref, bl_ref, g_ref, b_ref,
                      out_ref, n=n, np_=np_, final=True, wo_ref=wo_ref,
                      bo_ref=bo_ref)
    return pl.pallas_call(
        body, out_shape=jax.ShapeDtypeStruct((n, d), jnp.float32),
    )(r, p, inv, wl, bl.reshape(1, d), g.reshape(1, d), b.reshape(1, d),
      wo, bo.reshape(1, d))


def _layer1_call(r, p, cnt2d, wl, bl, g, b, n, d, np_):
    def body(r_ref, p_ref, cnt_ref, wl_ref, bl_ref, g_ref, b_ref, out_ref,
             inv_ref, hb_ref):
        cnt = jnp.maximum(cnt_ref[0:n] + cnt_ref[np_:np_ + n], 1.0)
        inv = 1.0 / cnt
        inv_ref[...] = inv
        agg = p_ref[0:n] + p_ref[np_:np_ + n]
        mean = agg * inv
        y = jnp.dot(mean, wl_ref[...], preferred_element_type=jnp.float32)
        y = y + bl_ref[...] + r_ref[...]
        mu = jnp.mean(y, axis=0, keepdims=True)
        var = jnp.mean((y - mu) * (y - mu), axis=0, keepdims=True)
        y = (y - mu) * lax.rsqrt(var + 1e-5) * g_ref[...] + b_ref[...]
        h = jnp.maximum(y, 0.0)
        out_ref[...] = h
        hb_ref[...] = h.astype(jnp.bfloat16)
    return pl.pallas_call(
        body,
        out_shape=[jax.ShapeDtypeStruct((n, d), jnp.float32),
                   jax.ShapeDtypeStruct((n, 1), jnp.float32),
                   jax.ShapeDtypeStruct((n, d), jnp.bfloat16)],
    )(r, p, cnt2d, wl, bl.reshape(1, d), g.reshape(1, d), b.reshape(1, d))


def kernel(x, edge_index, Wl, bl, Wr, gamma, beta, Wo, bo):
    src = edge_index[0]
    dst = edge_index[1]
    n, d = x.shape
    e = src.shape[0]
    L = Wl.shape[0]

    np_ = ((n + _LANES - 1) // _LANES) * _LANES
    chunk_edges = _CORES * _TILES * _LANES * _QROWS
    ep = ((e + chunk_edges - 1) // chunk_edges) * chunk_edges
    pad_e = ep - e
    if pad_e:
        if np_ == n:
            np_ += _LANES
        ar = jnp.arange(pad_e, dtype=jnp.int32)
        srcp = jnp.concatenate([src, (ar * 997) % n])
        dstp = jnp.concatenate([dst, n + ar % (np_ - n)])
    else:
        srcp, dstp = src, dst
    src2 = srcp.reshape(ep // _LANES, _LANES)
    dst2 = dstp.reshape(ep // (_LANES // 2), _LANES // 2)
    zer2 = jnp.zeros((np_, d), jnp.float32)
    zer1 = jnp.zeros((np_,), jnp.float32)
    ones1 = jnp.ones((_LANES // 2,), jnp.float32)

    cc = jnp.arange(d, dtype=jnp.int32)
    kk, tt = cc // 32, cc % 32
    orig = jnp.where(tt < 16, 32 * kk + 2 * tt, 32 * kk + 2 * (tt - 16) + 1)
    Wl_p = Wl[:, orig, :]

    agg_first = _make_sc_agg(n, d, np_, ep, with_cnt=True)
    agg_rest = _make_sc_agg(n, d, np_, ep, with_cnt=False)

    def i32view(hb):
        return lax.bitcast_convert_type(hb.reshape(n, d // 2, 2), jnp.int32)

    h = x
    hb = i32view(x.astype(jnp.bfloat16))
    inv = None
    for i in range(L):
        if i == 0:
            p, cntp = agg_first(hb, src2, dst2, zer2, zer1, ones1)
            r = _r_call(h, Wr[i])
            h, inv, hb = _layer1_call(r, p, cntp.reshape(_CORES * np_, 1),
                                      Wl_p[i], bl[i], gamma[i], beta[i],
                                      n, d, np_)
            hb = i32view(hb)
        elif i == L - 1:
            (p,) = agg_rest(hb, src2, dst2, zer2)
            r = _r_call(h, Wr[i])
            h = _final_layer_call(r, p, inv, Wl_p[i], bl[i], gamma[i],
                                  beta[i], Wo, bo, n, d, np_)
        else:
            (p,) = agg_rest(hb, src2, dst2, zer2)
            r = _r_call(h, Wr[i])
            h, hb = _mid_layer_call(r, p, inv, Wl_p[i], bl[i], gamma[i],
                                    beta[i], n, d, np_)
            hb = i32view(hb)
    return h


# speedup vs baseline: 2.2184x; 2.2184x over previous
import functools

import jax
import jax.numpy as jnp
from jax import lax
from jax.experimental import pallas as pl
from jax.experimental.pallas import tpu as pltpu
from jax.experimental.pallas import tpu_sc as plsc

_LANES = 128
_QROWS = 16
_TILES = 16
_CORES = 2


def _make_sc_agg(n, d, np_, ep, with_cnt):
    rows_per_tile = ep // (_CORES * _TILES) // _LANES
    slc = np_ // _TILES

    mesh = plsc.VectorSubcoreMesh(core_axis_name="c", subcore_axis_name="s")

    out_types = [jax.ShapeDtypeStruct((_CORES * np_, d), jnp.float32)]
    half = rows_per_tile // 2
    scratch = [
        pltpu.VMEM((half, _LANES), jnp.int32),
        pltpu.VMEM((half, _LANES), jnp.int32),
        pltpu.VMEM((2, _LANES, d), jnp.float32),
        pltpu.VMEM_SHARED((np_, d), jnp.float32),
        pltpu.SemaphoreType.DMA,
        pltpu.SemaphoreType.DMA,
    ]
    if with_cnt:
        out_types.append(jax.ShapeDtypeStruct((_CORES * np_,), jnp.float32))
        scratch += [
            pltpu.VMEM((_LANES,), jnp.float32),
            pltpu.VMEM_SHARED((np_,), jnp.float32),
            pltpu.VMEM((slc,), jnp.float32),
        ]

    def body(h, src2, dst2, zer2, zer1, ones1, p_out, cnt_out, sidx, didx,
             rows, acc, sem, sem_s, ones_v, cntacc, cz):
        c = lax.axis_index("c")
        s = lax.axis_index("s")
        base = s * slc
        pltpu.sync_copy(zer2.at[pl.ds(base, slc)], acc.at[pl.ds(base, slc)])
        if with_cnt:
            pltpu.sync_copy(zer1.at[pl.ds(base, slc)], cz)
            pltpu.sync_copy(cz, cntacc.at[pl.ds(base, slc)])
            pltpu.sync_copy(ones1, ones_v)
        wrow = (c * _TILES + s) * rows_per_tile
        plsc.subcore_barrier()

        for hh in range(2):
            hrow = wrow + hh * half
            pltpu.sync_copy(src2.at[pl.ds(hrow, half)], sidx)
            pltpu.sync_copy(dst2.at[pl.ds(hrow, half)], didx)
            pltpu.async_copy(h.at[sidx.at[0]], rows.at[0], sem)

            def group(j, par):
                @pl.when(j + 1 < half)
                def _():
                    pltpu.async_copy(h.at[sidx.at[j + 1]], rows.at[1 - par],
                                     sem)

                pltpu.make_async_copy(h.at[sidx.at[j]], rows.at[par],
                                      sem).wait()
                pltpu.sync_copy(rows.at[par], acc.at[didx.at[j]], add=True)
                if with_cnt:
                    pltpu.sync_copy(ones_v, cntacc.at[didx.at[j]], add=True)

            def jb2(jj, carry2):
                group(2 * jj, 0)
                group(2 * jj + 1, 1)
                return carry2

            lax.fori_loop(0, half // 2, jb2, 0)
        plsc.subcore_barrier()

        pltpu.sync_copy(acc.at[pl.ds(base, slc)],
                        p_out.at[pl.ds(c * np_ + base, slc)])
        if with_cnt:
            pltpu.sync_copy(cntacc.at[pl.ds(base, slc)], cz)
            pltpu.sync_copy(cz, cnt_out.at[pl.ds(c * np_ + base, slc)])

    if with_cnt:
        fn = body
    else:
        def fn(h, src2, dst2, zer2, p_out, sidx, didx, rows, acc, sem, sem_s):
            return body(h, src2, dst2, zer2, None, None, p_out, None, sidx,
                        didx, rows, acc, sem, sem_s, None, None, None)

    return pl.kernel(fn, mesh=mesh, out_type=out_types, scratch_types=scratch)


def _combine_body(h_ref, wr_ref, p_ref, inv_ref, wl_ref, bl_ref, g_ref,
                  b_ref, out_ref, *, n, np_, final, wo_ref=None, bo_ref=None):
    agg = p_ref[0:n] + p_ref[np_:np_ + n]
    mean = agg * inv_ref[...]
    y = jnp.dot(mean, wl_ref[...], preferred_element_type=jnp.float32)
    y = y + bl_ref[...]
    y = y + jnp.dot(h_ref[...], wr_ref[...], preferred_element_type=jnp.float32)
    mu = jnp.mean(y, axis=0, keepdims=True)
    var = jnp.mean((y - mu) * (y - mu), axis=0, keepdims=True)
    y = (y - mu) * lax.rsqrt(var + 1e-5) * g_ref[...] + b_ref[...]
    h = jnp.maximum(y, 0.0)
    if final:
        out_ref[...] = jnp.dot(h, wo_ref[...],
                               preferred_element_type=jnp.float32) + bo_ref[...]
    else:
        out_ref[...] = h


def _mid_layer_call(h, wr, p, inv, wl, bl, g, b, n, d, np_):
    def body(h_ref, wr_ref, p_ref, inv_ref, wl_ref, bl_ref, g_ref, b_ref,
             out_ref):
        _combine_body(h_ref, wr_ref, p_ref, inv_ref, wl_ref, bl_ref, g_ref,
                      b_ref, out_ref, n=n, np_=np_, final=False)
    return pl.pallas_call(
        body, out_shape=jax.ShapeDtypeStruct((n, d), jnp.float32),
    )(h, wr, p, inv, wl, bl.reshape(1, d), g.reshape(1, d), b.reshape(1, d))


def _final_layer_call(h, wr, p, inv, wl, bl, g, b, wo, bo, n, d, np_):
    def body(h_ref, wr_ref, p_ref, inv_ref, wl_ref, bl_ref, g_ref, b_ref,
             wo_ref, bo_ref, out_ref):
        _combine_body(h_ref, wr_ref, p_ref, inv_ref, wl_ref, bl_ref, g_ref,
                      b_ref, out_ref, n=n, np_=np_, final=True, wo_ref=wo_ref,
                      bo_ref=bo_ref)
    return pl.pallas_call(
        body, out_shape=jax.ShapeDtypeStruct((n, d), jnp.float32),
    )(h, wr, p, inv, wl, bl.reshape(1, d), g.reshape(1, d), b.reshape(1, d),
      wo, bo.reshape(1, d))


def _layer1_call(h, wr, p, cnt2d, wl, bl, g, b, n, d, np_):
    def body(h_ref, wr_ref, p_ref, cnt_ref, wl_ref, bl_ref, g_ref, b_ref,
             out_ref, inv_ref):
        cnt = jnp.maximum(cnt_ref[0:n] + cnt_ref[np_:np_ + n], 1.0)
        inv = 1.0 / cnt
        inv_ref[...] = inv
        agg = p_ref[0:n] + p_ref[np_:np_ + n]
        mean = agg * inv
        y = jnp.dot(mean, wl_ref[...], preferred_element_type=jnp.float32)
        y = y + bl_ref[...]
        y = y + jnp.dot(h_ref[...], wr_ref[...],
                        preferred_element_type=jnp.float32)
        mu = jnp.mean(y, axis=0, keepdims=True)
        var = jnp.mean((y - mu) * (y - mu), axis=0, keepdims=True)
        y = (y - mu) * lax.rsqrt(var + 1e-5) * g_ref[...] + b_ref[...]
        out_ref[...] = jnp.maximum(y, 0.0)
    return pl.pallas_call(
        body,
        out_shape=[jax.ShapeDtypeStruct((n, d), jnp.float32),
                   jax.ShapeDtypeStruct((n, 1), jnp.float32)],
    )(h, wr, p, cnt2d, wl, bl.reshape(1, d), g.reshape(1, d),
      b.reshape(1, d))


def kernel(x, edge_index, Wl, bl, Wr, gamma, beta, Wo, bo):
    src = edge_index[0]
    dst = edge_index[1]
    n, d = x.shape
    e = src.shape[0]
    L = Wl.shape[0]

    np_ = ((n + _LANES - 1) // _LANES) * _LANES
    chunk_edges = _CORES * _TILES * _LANES * _QROWS
    ep = ((e + chunk_edges - 1) // chunk_edges) * chunk_edges
    pad_e = ep - e
    if pad_e:
        if np_ == n:
            np_ += _LANES
        ar = jnp.arange(pad_e, dtype=jnp.int32)
        srcp = jnp.concatenate([src, (ar * 997) % n])
        dstp = jnp.concatenate([dst, n + ar % (np_ - n)])
    else:
        srcp, dstp = src, dst
    src2 = srcp.reshape(ep // _LANES, _LANES)
    dst2 = dstp.reshape(ep // _LANES, _LANES)
    zer2 = jnp.zeros((np_, d), jnp.float32)
    zer1 = jnp.zeros((np_,), jnp.float32)
    ones1 = jnp.ones((_LANES,), jnp.float32)

    agg_first = _make_sc_agg(n, d, np_, ep, with_cnt=True)
    agg_rest = _make_sc_agg(n, d, np_, ep, with_cnt=False)

    h = x
    inv = None
    for i in range(L):
        if i == 0:
            p, cntp = agg_first(h, src2, dst2, zer2, zer1, ones1)
            h, inv = _layer1_call(h, Wr[i], p, cntp.reshape(_CORES * np_, 1),
                                  Wl[i], bl[i], gamma[i], beta[i], n, d, np_)
        elif i == L - 1:
            (p,) = agg_rest(h, src2, dst2, zer2)
            h = _final_layer_call(h, Wr[i], p, inv, Wl[i], bl[i], gamma[i],
                                  beta[i], Wo, bo, n, d, np_)
        else:
            (p,) = agg_rest(h, src2, dst2, zer2)
            h = _mid_layer_call(h, Wr[i], p, inv, Wl[i], bl[i], gamma[i],
                                beta[i], n, d, np_)
    return h
